# Initial kernel scaffold; baseline (speedup 1.0000x reference)
import jax
import jax.numpy as jnp
from jax.experimental import pallas as pl


def kernel(x, h, g, edge_index, batch_ids, W1, b1, gamma, beta, W2, b2):
    raise NotImplementedError("write your pallas kernel here")


# speedup vs baseline: 10.1106x; 10.1106x over previous
import functools

import jax
import jax.numpy as jnp
from jax import lax
from jax.experimental import pallas as pl
from jax.experimental.pallas import tpu as pltpu
from jax.experimental.pallas import tpu_sc as plsc

N = 10000
NP = 10240
IN_DIM = 256
HID = 512
H2 = 1024
E2 = 320000
NUM_GRAPHS = 64

_NC, _NS = 2, 16
_NW = _NC * _NS
EPW = E2 // _NW

RA = 1024
RD = 1024

_INTERPRET = False


@functools.cache
def _sc_mesh():
    return plsc.VectorSubcoreMesh(
        core_axis_name="c", subcore_axis_name="s",
        num_cores=_NC, num_subcores=_NS)


def _row_sum(z):
    acc = z[:, 0:128]
    for c in range(1, 8):
        acc = acc + z[:, c * 128:(c + 1) * 128]
    g = acc[:, 0:8]
    for i in range(1, 16):
        g = g + acc[:, i * 8:(i + 1) * 8]
    b4 = g[:, 0:4] + g[:, 4:8]
    b2 = b4[:, 0:2] + b4[:, 2:4]
    return b2[:, 0:1] + b2[:, 1:2]


def _head_body(h_ref, w1_ref, b1_ref, gm_ref, bt_ref, w2_ref, b2_ref, o_ref,
               z_out, ln_out):
    z = lax.dot_general(h_ref[...].astype(jnp.bfloat16),
                        w1_ref[...].astype(jnp.bfloat16),
                        (((1,), (1,)), ((), ())),
                        preferred_element_type=jnp.float32)
    z = z + b1_ref[...][None, :]
    z_out[...] = z
    mu = _row_sum(z) * (1.0 / H2)
    var = _row_sum((z - mu) ** 2) * (1.0 / H2)
    z = (z - mu) / jnp.sqrt(var + 1e-5) * gm_ref[...][None, :] + bt_ref[...][None, :]
    zl = jnp.maximum(z, 0.0)
    ln_out[...] = zl
    y = lax.dot_general(zl.astype(jnp.bfloat16),
                        w2_ref[...].astype(jnp.bfloat16),
                        (((1,), (1,)), ((), ())),
                        preferred_element_type=jnp.float32)
    o_ref[...] = jax.nn.sigmoid(y[:, :1] + b2_ref[0])


def _head_call(h_pad, W1, b1, gamma, beta, W2, b2):
    W2p = jnp.pad(W2, ((0, 128 - W2.shape[0]), (0, 0)))
    return pl.pallas_call(
        _head_body,
        grid=(NP // RA,),
        in_specs=[
            pl.BlockSpec((RA, HID), lambda i: (i, 0)),
            pl.BlockSpec((H2, HID), lambda i: (0, 0)),
            pl.BlockSpec((H2,), lambda i: (0,)),
            pl.BlockSpec((H2,), lambda i: (0,)),
            pl.BlockSpec((H2,), lambda i: (0,)),
            pl.BlockSpec((128, H2), lambda i: (0, 0)),
            pl.BlockSpec(memory_space=pltpu.SMEM),
        ],
        out_specs=[pl.BlockSpec((RA, 1), lambda i: (i, 0)),
                   pl.BlockSpec((RA, H2), lambda i: (i, 0)),
                   pl.BlockSpec((RA, H2), lambda i: (i, 0))],
        out_shape=[jax.ShapeDtypeStruct((NP, 1), jnp.float32),
                   jax.ShapeDtypeStruct((NP, H2), jnp.float32),
                   jax.ShapeDtypeStruct((NP, H2), jnp.float32)],
        interpret=_INTERPRET,
    )(h_pad, W1, b1, gamma, beta, W2p, b2)[0]


def _edge_body(edge_hbm, out_hbm, idx_v, mask_v):
    wid = lax.axis_index("s") * _NC + lax.axis_index("c")
    pltpu.sync_copy(edge_hbm.at[pl.ds(wid * EPW, EPW)], idx_v)
    zeros16 = jnp.zeros((16,), jnp.int32)
    ones16 = jnp.ones((16,), jnp.int32)

    def zbody(c, carry):
        mask_v[pl.ds(c * 16, 16)] = zeros16
        return carry
    lax.fori_loop(0, NP // 16, zbody, 0)

    def sbody(c, carry):
        idx = idx_v[pl.ds(c * 16, 16)]
        plsc.store_scatter(mask_v, [idx], ones16)
        return carry
    lax.fori_loop(0, EPW // 16, sbody, 0)
    pltpu.sync_copy(mask_v, out_hbm.at[wid])


def _edge_call(edge_flat):
    return pl.kernel(
        _edge_body,
        out_type=jax.ShapeDtypeStruct((_NW, NP), jnp.int32),
        mesh=_sc_mesh(),
        scratch_types=[
            pltpu.VMEM((EPW,), jnp.int32),
            pltpu.VMEM((NP,), jnp.int32),
        ],
        compiler_params=pltpu.CompilerParams(needs_layout_passes=False),
        interpret=_INTERPRET,
    )(edge_flat)


def _topk_body(logits_hbm, batch_hbm, out_hbm, lg_v, bt_v, keep_v):
    wid = lax.axis_index("s") * _NC + lax.axis_index("c")
    pltpu.sync_copy(logits_hbm, lg_v.at[pl.ds(0, NP)])
    pltpu.sync_copy(batch_hbm, bt_v.at[pl.ds(0, NP)])
    zeros16 = jnp.zeros((16,), jnp.float32)

    def zbody(c, carry):
        keep_v[pl.ds(c * 16, 16)] = zeros16
        return carry
    lax.fori_loop(0, NP // 16, zbody, 0)

    lane = lax.iota(jnp.int32, 16)

    def lower_bound(tgt):
        def bbody(_, lh):
            lo, hi = lh
            mid = (lo + hi) // 2
            p = bt_v[pl.ds(mid, 16)][0] < tgt
            return (jnp.where(p, mid + 1, lo), jnp.where(p, hi, mid))
        lo, _ = lax.fori_loop(0, 14, bbody,
                              (jnp.int32(0), jnp.int32(NP)))
        return lo

    def do_graph(g):
        s = lower_bound(g)
        e = lower_bound(g + 1)
        n = e - s
        k = (9 * n + 9) // 10
        c0 = s // 16
        c1 = (e + 15) // 16

        def chunk_body(c, carry):
            base = c * 16
            gidx = base + lane
            li = lg_v[pl.ds(base, 16)]
            valid = (gidx >= s) & (gidx < e)

            def jbody(j, cnt):
                lj = jnp.full((16,), lg_v[pl.ds(j, 16)][0], jnp.float32)
                cond = (lj > li) | ((lj == li) & (j < gidx))
                return cnt + jnp.where(cond, 1, 0)
            cnt = lax.fori_loop(s, e, jbody, jnp.zeros((16,), jnp.int32))
            kv = jnp.where(cnt < k, 1.0, 0.0)
            gidx_safe = jnp.where(valid, gidx, NP + lane)
            plsc.store_scatter(keep_v, [gidx_safe], kv)
            return carry
        lax.fori_loop(c0, c1, chunk_body, 0)

    do_graph(wid * 2)
    do_graph(wid * 2 + 1)
    pltpu.sync_copy(keep_v.at[pl.ds(0, NP)], out_hbm.at[wid])


def _topk_call(logits_flat, batch_pad):
    return pl.kernel(
        _topk_body,
        out_type=jax.ShapeDtypeStruct((_NW, NP), jnp.float32),
        mesh=_sc_mesh(),
        scratch_types=[
            pltpu.VMEM((NP + 16,), jnp.float32),
            pltpu.VMEM((NP + 16,), jnp.int32),
            pltpu.VMEM((NP + 16,), jnp.float32),
        ],
        compiler_params=pltpu.CompilerParams(needs_layout_passes=False),
        interpret=_INTERPRET,
    )(logits_flat, batch_pad)


def _finish_body(x_ref, n_ref, l_ref, kp_ref, ep_ref, xo_ref, eo_ref):
    keep = jnp.sum(kp_ref[...], axis=0).reshape(RD, 1)
    lg = l_ref[...]
    noise = n_ref[...] * 0.5 + 0.5
    xo_ref[...] = keep * (x_ref[...] * lg) + (1.0 - keep) * noise
    em = jnp.sum(ep_ref[...], axis=0).reshape(RD, 1)
    eo_ref[...] = (em > 0).astype(jnp.int32)


def _finish_call(x_pad, noise_raw, logits2d, keepP, edgeP):
    return pl.pallas_call(
        _finish_body,
        grid=(NP // RD,),
        in_specs=[
            pl.BlockSpec((RD, IN_DIM), lambda i: (i, 0)),
            pl.BlockSpec((RD, IN_DIM), lambda i: (i, 0)),
            pl.BlockSpec((RD, 1), lambda i: (i, 0)),
            pl.BlockSpec((_NW, RD), lambda i: (0, i)),
            pl.BlockSpec((_NW, RD), lambda i: (0, i)),
        ],
        out_specs=[
            pl.BlockSpec((RD, IN_DIM), lambda i: (i, 0)),
            pl.BlockSpec((RD, 1), lambda i: (i, 0)),
        ],
        out_shape=[
            jax.ShapeDtypeStruct((NP, IN_DIM), jnp.float32),
            jax.ShapeDtypeStruct((NP, 1), jnp.int32),
        ],
        interpret=_INTERPRET,
    )(x_pad, noise_raw, logits2d, keepP, edgeP)


def kernel(x, h, g, edge_index, batch_ids, W1, b1, gamma, beta, W2, b2):
    del g
    h_pad = jnp.pad(h, ((0, NP - N), (0, 0)))
    x_pad = jnp.pad(x, ((0, NP - N), (0, 0)))
    batch_pad = jnp.pad(batch_ids, (0, NP - N), constant_values=NUM_GRAPHS)
    noise_raw = jax.random.normal(jax.random.key(42), (N, IN_DIM), jnp.float32)
    noise_pad = jnp.pad(noise_raw, ((0, NP - N), (0, 0)))
    edge_flat = edge_index.reshape(-1)

    logits2d = _head_call(h_pad, W1, b1, gamma, beta, W2, b2)
    edgeP = _edge_call(edge_flat)
    keepP = _topk_call(logits2d.reshape(NP), batch_pad)
    x_new_pad, edge2d = _finish_call(x_pad, noise_pad, logits2d, keepP, edgeP)

    x_new = x_new_pad[:N]
    node_weight = logits2d[:N, 0]
    edge_mask = edge2d[:N, 0] != 0
    return (x_new, node_weight, edge_mask)
